# Initial kernel scaffold; baseline (speedup 1.0000x reference)
import jax
import jax.numpy as jnp
from jax.experimental import pallas as pl


def kernel(object_ids, embedding_weight):
    raise NotImplementedError("write your pallas kernel here")


# speedup vs baseline: 9.2309x; 9.2309x over previous
import functools

import jax
import jax.numpy as jnp
from jax import lax
from jax.experimental import pallas as pl
from jax.experimental.pallas import tpu as pltpu
from jax.experimental.pallas import tpu_sc as plsc

MAX_ID = 100000
D_MODEL = 128
B = 4096
L = 200

NW = 32
K = 128
N_TOTAL = B * L
CHUNKS_PER_W = N_TOTAL // (NW * K)
NBUF = 4


def _sc_gather(ids2d, table):
    mesh = plsc.VectorSubcoreMesh(core_axis_name="c", subcore_axis_name="s")

    @functools.partial(
        pl.kernel,
        out_type=jax.ShapeDtypeStruct((N_TOTAL, D_MODEL), jnp.float32),
        mesh=mesh,
        scratch_types=(
            pltpu.VMEM((CHUNKS_PER_W, K), jnp.int32),
            [pltpu.VMEM((K, D_MODEL), jnp.float32) for _ in range(NBUF)],
            [pltpu.SemaphoreType.DMA for _ in range(NBUF)],
            [pltpu.SemaphoreType.DMA for _ in range(NBUF)],
        ),
    )
    def k(ids_hbm, table_hbm, out_hbm, idx_v, bufs, gsems, wsems):
        wid = lax.axis_index("s") * 2 + lax.axis_index("c")
        row0 = wid * CHUNKS_PER_W

        pltpu.sync_copy(ids_hbm.at[pl.ds(row0, CHUNKS_PER_W)], idx_v)

        def fire_gather(i, b):
            pltpu.async_copy(table_hbm.at[idx_v.at[i]], bufs[b], gsems[b])

        def wait_gather(i, b):
            pltpu.make_async_copy(table_hbm.at[idx_v.at[i]], bufs[b],
                                  gsems[b]).wait()

        def fire_write(i, b):
            dst = out_hbm.at[pl.ds((row0 + i) * K, K)]
            pltpu.async_copy(bufs[b], dst, wsems[b])

        def wait_write(i, b):
            dst = out_hbm.at[pl.ds((row0 + i) * K, K)]
            pltpu.make_async_copy(bufs[b], dst, wsems[b]).wait()

        for b in range(NBUF):
            fire_gather(b, b)

        @pl.loop(0, CHUNKS_PER_W - NBUF, step=NBUF)
        def _(g):
            for b in range(NBUF):
                i = g + b
                wait_gather(i, b)
                fire_write(i, b)
                wait_write(i, b)
                fire_gather(i + NBUF, b)

        for b in range(NBUF):
            i = CHUNKS_PER_W - NBUF + b
            wait_gather(i, b)
            fire_write(i, b)
            wait_write(i, b)

    return k(ids2d, table)


def kernel(object_ids, embedding_weight):
    ids2d = object_ids.astype(jnp.int32).reshape(N_TOTAL // K, K)
    out = _sc_gather(ids2d, embedding_weight)
    return out.reshape(B, L, D_MODEL)
